# Initial kernel scaffold; baseline (speedup 1.0000x reference)
import jax
import jax.numpy as jnp
from jax.experimental import pallas as pl


def kernel(xo, xc, ctx_g1, ctx_b1, ctx_W1, ctx_c1, ctx_g2, ctx_b2, ctx_W2, ctx_c2, obj_g1, obj_b1, obj_W1, obj_c1, obj_g2, obj_b2, obj_W2, obj_c2, rnd_g1, rnd_b1, rnd_W1, rnd_c1, rnd_g2, rnd_b2, rnd_W2, rnd_c2):
    raise NotImplementedError("write your pallas kernel here")


# speedup vs baseline: 2.6882x; 2.6882x over previous
import functools

import jax
import jax.numpy as jnp
from jax.experimental import pallas as pl

_EPS = 1e-5


def _stats_kernel(xo_ref, xc_ref, out_ref):
    j = pl.program_id(0)
    xo = xo_ref[...]
    xc = xc_ref[...]
    rows = [
        jnp.sum(xo, axis=0, keepdims=True),
        jnp.sum(xo * xo, axis=0, keepdims=True),
        jnp.sum(xc, axis=0, keepdims=True),
        jnp.sum(xc * xc, axis=0, keepdims=True),
        jnp.sum(xo * xc, axis=0, keepdims=True),
    ]
    block = jnp.concatenate(rows + [jnp.zeros((3, xo.shape[1]), jnp.float32)], axis=0)

    @pl.when(j == 0)
    def _():
        out_ref[...] = block

    @pl.when(j > 0)
    def _():
        out_ref[...] += block


def _hstats_kernel(xo_ref, xc_ref, wc_ref, cc_ref, wo_ref, co_ref, wr_ref, cr_ref,
                   out_ref):
    j = pl.program_id(0)
    xo = xo_ref[...]
    xc = xc_ref[...]
    xr = xo + xc
    rows = []
    for x, w_ref, c_ref in ((xc, wc_ref, cc_ref), (xo, wo_ref, co_ref),
                            (xr, wr_ref, cr_ref)):
        h = jnp.maximum(
            jnp.dot(x, w_ref[...], preferred_element_type=jnp.float32)
            + c_ref[...], 0.0)
        rows.append(jnp.sum(h, axis=0, keepdims=True))
        rows.append(jnp.sum(h * h, axis=0, keepdims=True))
    block = jnp.concatenate(
        rows + [jnp.zeros((2, xo.shape[1]), jnp.float32)], axis=0)

    @pl.when(j == 0)
    def _():
        out_ref[...] = block

    @pl.when(j > 0)
    def _():
        out_ref[...] += block


def _final_kernel(xo_ref, xc_ref,
                  wc1_ref, cc1_ref, wo1_ref, co1_ref, wr1_ref, cr1_ref,
                  wc2_ref, cc2_ref, wo2_ref, co2_ref, wr2_ref, cr2_ref,
                  oc_ref, oo_ref, or_ref):
    xo = xo_ref[...]
    xc = xc_ref[...]
    xr = xo + xc

    def head(x, w1_ref, c1_ref, w2_ref, c2_ref):
        h = jnp.maximum(
            jnp.dot(x, w1_ref[...], preferred_element_type=jnp.float32)
            + c1_ref[...], 0.0)
        return (jnp.dot(h, w2_ref[...], preferred_element_type=jnp.float32)
                + c2_ref[...])

    def log_softmax(z):
        m = jnp.max(z, axis=-1, keepdims=True)
        s = z - m
        return s - jnp.log(jnp.sum(jnp.exp(s), axis=-1, keepdims=True))

    oc_ref[...] = log_softmax(head(xc, wc1_ref, cc1_ref, wc2_ref, cc2_ref))
    oo_ref[...] = head(xo, wo1_ref, co1_ref, wo2_ref, co2_ref)
    or_ref[...] = log_softmax(head(xr, wr1_ref, cr1_ref, wr2_ref, cr2_ref))


def _row_spec(r, h):
    return pl.BlockSpec((r, h), lambda j: (j, 0))


def _rep_spec(shape):
    return pl.BlockSpec(shape, lambda j: tuple(0 for _ in shape))


def _fold1(m, v, g, b, W, c):
    a = g / jnp.sqrt(v + _EPS)
    return a[:, None] * W, ((b - a * m)[None, :] @ W) + c[None, :]


@functools.partial(jax.jit, static_argnames=())
def kernel(xo, xc,
           ctx_g1, ctx_b1, ctx_W1, ctx_c1, ctx_g2, ctx_b2, ctx_W2, ctx_c2,
           obj_g1, obj_b1, obj_W1, obj_c1, obj_g2, obj_b2, obj_W2, obj_c2,
           rnd_g1, rnd_b1, rnd_W1, rnd_c1, rnd_g2, rnd_b2, rnd_W2, rnd_c2):
    B, H = xo.shape
    O = ctx_W2.shape[1]
    R = 2000 if B % 2000 == 0 else (1000 if B % 1000 == 0 else B)
    nb = B // R

    stats = pl.pallas_call(
        _stats_kernel,
        grid=(nb,),
        in_specs=[_row_spec(R, H), _row_spec(R, H)],
        out_specs=_rep_spec((8, H)),
        out_shape=jax.ShapeDtypeStruct((8, H), jnp.float32),
    )(xo, xc)

    inv_b = 1.0 / B
    m_xo = stats[0] * inv_b
    v_xo = stats[1] * inv_b - m_xo * m_xo
    m_xc = stats[2] * inv_b
    v_xc = stats[3] * inv_b - m_xc * m_xc
    m_xr = m_xo + m_xc
    v_xr = (stats[1] + stats[3] + 2.0 * stats[4]) * inv_b - m_xr * m_xr

    wc1, cc1 = _fold1(m_xc, v_xc, ctx_g1, ctx_b1, ctx_W1, ctx_c1)
    wo1, co1 = _fold1(m_xo, v_xo, obj_g1, obj_b1, obj_W1, obj_c1)
    wr1, cr1 = _fold1(m_xr, v_xr, rnd_g1, rnd_b1, rnd_W1, rnd_c1)

    hstats = pl.pallas_call(
        _hstats_kernel,
        grid=(nb,),
        in_specs=[_row_spec(R, H), _row_spec(R, H),
                  _rep_spec((H, H)), _rep_spec((1, H)),
                  _rep_spec((H, H)), _rep_spec((1, H)),
                  _rep_spec((H, H)), _rep_spec((1, H))],
        out_specs=_rep_spec((8, H)),
        out_shape=jax.ShapeDtypeStruct((8, H), jnp.float32),
    )(xo, xc, wc1, cc1, wo1, co1, wr1, cr1)

    m_hc = hstats[0] * inv_b
    v_hc = hstats[1] * inv_b - m_hc * m_hc
    m_ho = hstats[2] * inv_b
    v_ho = hstats[3] * inv_b - m_ho * m_ho
    m_hr = hstats[4] * inv_b
    v_hr = hstats[5] * inv_b - m_hr * m_hr

    wc2, cc2 = _fold1(m_hc, v_hc, ctx_g2, ctx_b2, ctx_W2, ctx_c2)
    wo2, co2 = _fold1(m_ho, v_ho, obj_g2, obj_b2, obj_W2, obj_c2)
    wr2, cr2 = _fold1(m_hr, v_hr, rnd_g2, rnd_b2, rnd_W2, rnd_c2)

    outs = pl.pallas_call(
        _final_kernel,
        grid=(nb,),
        in_specs=[_row_spec(R, H), _row_spec(R, H),
                  _rep_spec((H, H)), _rep_spec((1, H)),
                  _rep_spec((H, H)), _rep_spec((1, H)),
                  _rep_spec((H, H)), _rep_spec((1, H)),
                  _rep_spec((H, O)), _rep_spec((1, O)),
                  _rep_spec((H, O)), _rep_spec((1, O)),
                  _rep_spec((H, O)), _rep_spec((1, O))],
        out_specs=[_row_spec(R, O), _row_spec(R, O), _row_spec(R, O)],
        out_shape=[jax.ShapeDtypeStruct((B, O), jnp.float32)] * 3,
    )(xo, xc, wc1, cc1, wo1, co1, wr1, cr1,
      wc2, cc2, wo2, co2, wr2, cr2)

    return tuple(outs)
